# baseline (speedup 1.0000x reference)
import functools
import math

import numpy as np

import jax
import jax.numpy as jnp
from jax import lax
from jax.experimental import pallas as pl
from jax.experimental.pallas import tpu as pltpu
from jax.experimental.pallas import tpu_sc as plsc

N = 10000
E = 320000
D = 128
D_ATTR = 16
D_EMB = 16
FC_HIDDEN = 8

NUM_SC = 2
NUM_TILES = 16
NUM_WORKERS = NUM_SC * NUM_TILES

NSLICE = 5
ES = E // NSLICE
PER_WS = ES // NUM_WORKERS
CHUNK = 80
NCHUNK_S = PER_WS // CHUNK
K = 5
NGROUP_S = NCHUNK_S // K
K2 = 4

NP = 10240
STRIPE = NP // NUM_TILES

_INV_SQRT_D = 1.0 / math.sqrt(D)
_INV_SQRT_DA = 1.0 / math.sqrt(D_ATTR)
_INV_SQRT_DE = 1.0 / math.sqrt(D_EMB)
_INV_SQRT_FC = 1.0 / math.sqrt(FC_HIDDEN)
_INV_SQRT_NEIGH = 1.0 / math.sqrt(32.0)
_INV_SQRT_DDA = 1.0 / math.sqrt(D * D_ATTR)

_MESH = plsc.VectorSubcoreMesh(
    core_axis_name="c", subcore_axis_name="s",
    num_cores=NUM_SC, num_subcores=NUM_TILES)


def _node_prep_body(nf_ref, na_ref, w1_ref, wtpt_ref, wsct_ref,
                    x_ref, natp_ref, sc_ref):
    nf = nf_ref[...]
    na = na_ref[...]
    x_ref[...] = jnp.dot(nf, w1_ref[...],
                         preferred_element_type=jnp.float32) * _INV_SQRT_D
    natp_ref[...] = jnp.dot(na, wtpt_ref[...],
                            preferred_element_type=jnp.float32) * _INV_SQRT_DA
    acc = jnp.zeros_like(nf)
    for j in range(D_ATTR):
        acc = acc + jnp.dot(nf * na[:, j:j + 1], wsct_ref[j],
                            preferred_element_type=jnp.float32)
    sc_ref[...] = acc * _INV_SQRT_DDA


def _node_prep(nf, na, w1, wtpt, wsct):
    blk = 400
    grid = (N // blk,)
    return pl.pallas_call(
        _node_prep_body,
        grid=grid,
        in_specs=[
            pl.BlockSpec((blk, D), lambda i: (i, 0)),
            pl.BlockSpec((blk, D_ATTR), lambda i: (i, 0)),
            pl.BlockSpec((D, D), lambda i: (0, 0)),
            pl.BlockSpec((D_ATTR, D), lambda i: (0, 0)),
            pl.BlockSpec((D_ATTR, D, D), lambda i: (0, 0, 0)),
        ],
        out_specs=[
            pl.BlockSpec((blk, D), lambda i: (i, 0)),
            pl.BlockSpec((blk, D), lambda i: (i, 0)),
            pl.BlockSpec((blk, D), lambda i: (i, 0)),
        ],
        out_shape=[
            jax.ShapeDtypeStruct((N, D), jnp.float32),
            jax.ShapeDtypeStruct((N, D), jnp.float32),
            jax.ShapeDtypeStruct((N, D), jnp.float32),
        ],
    )(nf, na, w1, wtpt, wsct)


@functools.partial(
    pl.kernel,
    out_type=jax.ShapeDtypeStruct((ES, D), jnp.float32),
    mesh=_MESH,
    scratch_types=[
        pltpu.VMEM((PER_WS,), jnp.int32),
        pltpu.VMEM((K, CHUNK, D), jnp.float32),
        pltpu.SemaphoreType.DMA,
        pltpu.SemaphoreType.DMA,
    ],
)
def _sc_gather(x_hbm, src_hbm, out_hbm, idx_v, bufs_v, gsem, wsem):
    wid = lax.axis_index("c") * NUM_TILES + lax.axis_index("s")
    base = wid * PER_WS
    pltpu.sync_copy(src_hbm.at[pl.ds(base, PER_WS)], idx_v)

    @pl.loop(0, NGROUP_S)
    def _(g):
        goff = g * (K * CHUNK)
        hs = [
            pltpu.async_copy(
                x_hbm.at[idx_v.at[pl.ds(goff + b * CHUNK, CHUNK)]],
                bufs_v.at[b], gsem)
            for b in range(K)
        ]
        for h in hs:
            h.wait()
        ws = [
            pltpu.async_copy(
                bufs_v.at[b],
                out_hbm.at[pl.ds(base + goff + b * CHUNK, CHUNK)], wsem)
            for b in range(K)
        ]
        for h in ws:
            h.wait()


EBLK = 3200
RPB = EBLK // 8


def _edge_body(g_ref, emb_ref, eap_ref, bd1_ref, exp8_ref, bd2_ref, w2_ref,
               msg_ref):
    hp = jax.nn.silu(jnp.dot(emb_ref[...], bd1_ref[...],
                             preferred_element_type=jnp.float32)
                     * _INV_SQRT_DE)
    eap64 = jnp.dot(eap_ref[...], exp8_ref[...],
                    preferred_element_type=jnp.float32)
    cq = jnp.dot(hp * eap64, bd2_ref[...],
                 preferred_element_type=jnp.float32) * _INV_SQRT_FC
    for k in range(8):
        ck = cq[:, 128 * k:128 * (k + 1)]
        mk = g_ref[RPB * k:RPB * (k + 1), :] * ck
        msg_ref[RPB * k:RPB * (k + 1), :] = jax.nn.silu(
            jnp.dot(mk, w2_ref[...],
                    preferred_element_type=jnp.float32) * _INV_SQRT_D)


def _edge_stage(g, emb, eap8, bd1, exp8, bd2, w2, s):
    grid = (ES // EBLK,)
    nblk = ES // EBLK
    return pl.pallas_call(
        _edge_body,
        grid=grid,
        in_specs=[
            pl.BlockSpec((EBLK, D), lambda i: (i, 0)),
            pl.BlockSpec((RPB, 8 * D_EMB), lambda i: (i, 0)),
            pl.BlockSpec((RPB, 8), lambda i, s=s: (i + s * nblk, 0)),
            pl.BlockSpec((8 * D_EMB, 8 * FC_HIDDEN), lambda i: (0, 0)),
            pl.BlockSpec((8, 8 * FC_HIDDEN), lambda i: (0, 0)),
            pl.BlockSpec((8 * FC_HIDDEN, 8 * D), lambda i: (0, 0)),
            pl.BlockSpec((D, D), lambda i: (0, 0)),
        ],
        out_specs=pl.BlockSpec((EBLK, D), lambda i: (i, 0)),
        out_shape=jax.ShapeDtypeStruct((ES, D), jnp.float32),
    )(g, emb, eap8, bd1, exp8, bd2, w2)


_PBLK = np.arange(EBLK).reshape(RPB, 8).T.reshape(-1)
_EPERM = (np.arange(0, E, EBLK)[:, None] + _PBLK[None, :]).reshape(-1)


def _scatter_slices(nslices):
    n_msg = nslices

    @functools.partial(
        pl.kernel,
        out_type=jax.ShapeDtypeStruct((NUM_SC, NP, D), jnp.float32),
        mesh=_MESH,
        scratch_types=[
            pltpu.VMEM((CHUNK,), jnp.int32),
            pltpu.VMEM((CHUNK,), jnp.int32),
            pltpu.VMEM((CHUNK,), jnp.int32),
            pltpu.VMEM((CHUNK,), jnp.int32),
            pltpu.VMEM((K2, CHUNK, D), jnp.float32),
            pltpu.VMEM_SHARED((NP, D), jnp.float32),
            pltpu.SemaphoreType.DMA,
        ],
    )
    def kfn(*refs):
        msg_refs = refs[:n_msg]
        dst_refs = refs[n_msg:2 * n_msg]
        zeros_hbm = refs[2 * n_msg]
        out_hbm = refs[2 * n_msg + 1]
        i0, i1, i2, i3, rows_v, acc_sh, lsem = refs[2 * n_msg + 2:]
        idx_bufs = (i0, i1, i2, i3)

        cid = lax.axis_index("c")
        sid = lax.axis_index("s")
        wid = cid * NUM_TILES + sid
        base = wid * PER_WS

        pltpu.sync_copy(zeros_hbm, acc_sh.at[pl.ds(sid * STRIPE, STRIPE)])
        plsc.subcore_barrier()

        ngroup = NCHUNK_S // K2
        tail = NCHUNK_S - ngroup * K2

        for msg_hbm, dst_hbm in zip(msg_refs, dst_refs):
            @pl.loop(0, ngroup)
            def _(g):
                goff = base + g * (K2 * CHUNK)
                hs = []
                for b in range(K2):
                    hs.append(pltpu.async_copy(
                        dst_hbm.at[pl.ds(goff + b * CHUNK, CHUNK)],
                        idx_bufs[b], lsem))
                    hs.append(pltpu.async_copy(
                        msg_hbm.at[pl.ds(goff + b * CHUNK, CHUNK)],
                        rows_v.at[b], lsem))
                for h in hs:
                    h.wait()
                for b in range(K2):
                    pltpu.sync_copy(rows_v.at[b], acc_sh.at[idx_bufs[b]],
                                    add=True)

            for t in range(tail):
                toff = base + (ngroup * K2 + t) * CHUNK
                pltpu.sync_copy(dst_hbm.at[pl.ds(toff, CHUNK)], i0)
                pltpu.sync_copy(msg_hbm.at[pl.ds(toff, CHUNK)], rows_v.at[0])
                pltpu.sync_copy(rows_v.at[0], acc_sh.at[i0], add=True)

        plsc.subcore_barrier()
        pltpu.sync_copy(acc_sh.at[pl.ds(sid * STRIPE, STRIPE)],
                        out_hbm.at[cid, pl.ds(sid * STRIPE, STRIPE)])

    return kfn


_sc_scatter2 = _scatter_slices(2)
_sc_scatter3 = _scatter_slices(3)


def _final_body(pa_ref, pb_ref, natp_ref, sc_ref, w3_ref, out_ref):
    agg = (pa_ref[0] + pa_ref[1] + pb_ref[0] + pb_ref[1]) * _INV_SQRT_NEIGH
    upd = jax.nn.silu(
        jnp.dot(agg * natp_ref[...], w3_ref[...],
                preferred_element_type=jnp.float32) * _INV_SQRT_D)
    out_ref[...] = upd + sc_ref[...]


def _final(pa, pb, natp, sc, w3):
    blk = 400
    grid = (N // blk,)
    return pl.pallas_call(
        _final_body,
        grid=grid,
        in_specs=[
            pl.BlockSpec((NUM_SC, blk, D), lambda i: (0, i, 0)),
            pl.BlockSpec((NUM_SC, blk, D), lambda i: (0, i, 0)),
            pl.BlockSpec((blk, D), lambda i: (i, 0)),
            pl.BlockSpec((blk, D), lambda i: (i, 0)),
            pl.BlockSpec((D, D), lambda i: (0, 0)),
        ],
        out_specs=pl.BlockSpec((blk, D), lambda i: (i, 0)),
        out_shape=jax.ShapeDtypeStruct((N, D), jnp.float32),
    )(pa, pb, natp, sc, w3)


def kernel(node_feats, node_attrs, edge_embedding, edge_attrs, edge_index,
           W1, M1, M2, W2, Wtp, W3, Wsc):
    perm = jnp.asarray(_EPERM)
    src = jnp.take(edge_index[0], perm)
    dst = jnp.take(edge_index[1], perm)
    wtpt = Wtp.T
    wsct = jnp.transpose(Wsc, (2, 1, 0))

    eap8 = edge_attrs.reshape(E // 8, 8)
    eye8 = jnp.eye(8, dtype=jnp.float32)
    bd1 = (eye8[:, None, :, None] * M1[None, :, None, :]).reshape(
        8 * D_EMB, 8 * FC_HIDDEN)
    exp8 = (eye8[:, :, None] * jnp.ones((1, 1, FC_HIDDEN))).reshape(
        8, 8 * FC_HIDDEN)
    bd2 = (eye8[:, None, :, None] * M2[None, :, None, :]).reshape(
        8 * FC_HIDDEN, 8 * D)

    x, natp, sc = _node_prep(node_feats, node_attrs, W1, wtpt, wsct)

    msgs = []
    dsts = []
    for s in range(NSLICE):
        src_s = lax.slice_in_dim(src, s * ES, (s + 1) * ES)
        dst_s = lax.slice_in_dim(dst, s * ES, (s + 1) * ES)
        embp_s = lax.slice_in_dim(edge_embedding, s * ES,
                                  (s + 1) * ES).reshape(ES // 8, 8 * D_EMB)
        g_s = _sc_gather(x, src_s)
        msgs.append(_edge_stage(g_s, embp_s, eap8, bd1, exp8, bd2, W2, s))
        dsts.append(dst_s)

    zeros = jnp.zeros((STRIPE, D), jnp.float32)
    pa = _sc_scatter2(msgs[0], msgs[1], dsts[0], dsts[1], zeros)
    pb = _sc_scatter3(msgs[2], msgs[3], msgs[4], dsts[2], dsts[3], dsts[4],
                      zeros)
    return _final(pa, pb, natp, sc, W3)


# speedup vs baseline: 1.1489x; 1.1489x over previous
import functools
import math

import numpy as np

import jax
import jax.numpy as jnp
from jax import lax
from jax.experimental import pallas as pl
from jax.experimental.pallas import tpu as pltpu
from jax.experimental.pallas import tpu_sc as plsc

N = 10000
E = 320000
D = 128
D_ATTR = 16
D_EMB = 16
FC_HIDDEN = 8

NUM_SC = 2
NUM_TILES = 16
NUM_WORKERS = NUM_SC * NUM_TILES

PER_W = E // NUM_WORKERS
CHUNK = 80
NCHUNK = PER_W // CHUNK
K = 5
NGROUP = NCHUNK // K
K2 = 4

EA_EDGES = 192000
EB_EDGES = 128000

NP = 10240
STRIPE = NP // NUM_TILES

_INV_SQRT_D = 1.0 / math.sqrt(D)
_INV_SQRT_DA = 1.0 / math.sqrt(D_ATTR)
_INV_SQRT_DE = 1.0 / math.sqrt(D_EMB)
_INV_SQRT_FC = 1.0 / math.sqrt(FC_HIDDEN)
_INV_SQRT_NEIGH = 1.0 / math.sqrt(32.0)
_INV_SQRT_DDA = 1.0 / math.sqrt(D * D_ATTR)

_MESH = plsc.VectorSubcoreMesh(
    core_axis_name="c", subcore_axis_name="s",
    num_cores=NUM_SC, num_subcores=NUM_TILES)


def _node_prep_body(nf_ref, na_ref, w1_ref, wtpt_ref, wsct_ref,
                    x_ref, natp_ref, sc_ref):
    nf = nf_ref[...]
    na = na_ref[...]
    x_ref[...] = jnp.dot(nf, w1_ref[...],
                         preferred_element_type=jnp.float32) * _INV_SQRT_D
    natp_ref[...] = jnp.dot(na, wtpt_ref[...],
                            preferred_element_type=jnp.float32) * _INV_SQRT_DA
    acc = jnp.zeros_like(nf)
    for j in range(D_ATTR):
        acc = acc + jnp.dot(nf * na[:, j:j + 1], wsct_ref[j],
                            preferred_element_type=jnp.float32)
    sc_ref[...] = acc * _INV_SQRT_DDA


def _node_prep(nf, na, w1, wtpt, wsct):
    blk = 400
    grid = (N // blk,)
    return pl.pallas_call(
        _node_prep_body,
        grid=grid,
        in_specs=[
            pl.BlockSpec((blk, D), lambda i: (i, 0)),
            pl.BlockSpec((blk, D_ATTR), lambda i: (i, 0)),
            pl.BlockSpec((D, D), lambda i: (0, 0)),
            pl.BlockSpec((D_ATTR, D), lambda i: (0, 0)),
            pl.BlockSpec((D_ATTR, D, D), lambda i: (0, 0, 0)),
        ],
        out_specs=[
            pl.BlockSpec((blk, D), lambda i: (i, 0)),
            pl.BlockSpec((blk, D), lambda i: (i, 0)),
            pl.BlockSpec((blk, D), lambda i: (i, 0)),
        ],
        out_shape=[
            jax.ShapeDtypeStruct((N, D), jnp.float32),
            jax.ShapeDtypeStruct((N, D), jnp.float32),
            jax.ShapeDtypeStruct((N, D), jnp.float32),
        ],
    )(nf, na, w1, wtpt, wsct)


@functools.partial(
    pl.kernel,
    out_type=jax.ShapeDtypeStruct((E, D), jnp.float32),
    mesh=_MESH,
    scratch_types=[
        pltpu.VMEM((PER_W,), jnp.int32),
        pltpu.VMEM((K, CHUNK, D), jnp.float32),
        pltpu.SemaphoreType.DMA,
        pltpu.SemaphoreType.DMA,
    ],
)
def _sc_gather(x_hbm, src_hbm, out_hbm, idx_v, bufs_v, gsem, wsem):
    wid = lax.axis_index("c") * NUM_TILES + lax.axis_index("s")
    base = wid * PER_W
    pltpu.sync_copy(src_hbm.at[pl.ds(base, PER_W)], idx_v)

    @pl.loop(0, NGROUP)
    def _(g):
        goff = g * (K * CHUNK)
        hs = [
            pltpu.async_copy(
                x_hbm.at[idx_v.at[pl.ds(goff + b * CHUNK, CHUNK)]],
                bufs_v.at[b], gsem)
            for b in range(K)
        ]
        for h in hs:
            h.wait()
        ws = [
            pltpu.async_copy(
                bufs_v.at[b],
                out_hbm.at[pl.ds(base + goff + b * CHUNK, CHUNK)], wsem)
            for b in range(K)
        ]
        for h in ws:
            h.wait()


EBLK = 3200
RPB = EBLK // 8


def _edge_body(g_ref, emb_ref, eap_ref, bd1_ref, exp8_ref, bd2_ref, w2_ref,
               msg_ref):
    hp = jax.nn.silu(jnp.dot(emb_ref[...], bd1_ref[...],
                             preferred_element_type=jnp.float32)
                     * _INV_SQRT_DE)
    eap64 = jnp.dot(eap_ref[...], exp8_ref[...],
                    preferred_element_type=jnp.float32)
    cq = jnp.dot(hp * eap64, bd2_ref[...],
                 preferred_element_type=jnp.float32) * _INV_SQRT_FC
    for k in range(8):
        ck = cq[:, 128 * k:128 * (k + 1)]
        mk = g_ref[RPB * k:RPB * (k + 1), :] * ck
        msg_ref[RPB * k:RPB * (k + 1), :] = jax.nn.silu(
            jnp.dot(mk, w2_ref[...],
                    preferred_element_type=jnp.float32) * _INV_SQRT_D)


def _edge_stage(g, embp, eap8, bd1, exp8, bd2, w2, blk0, nblk):
    return pl.pallas_call(
        _edge_body,
        grid=(nblk,),
        in_specs=[
            pl.BlockSpec((EBLK, D), lambda i: (i + blk0, 0)),
            pl.BlockSpec((RPB, 8 * D_EMB), lambda i: (i + blk0, 0)),
            pl.BlockSpec((RPB, 8), lambda i: (i + blk0, 0)),
            pl.BlockSpec((8 * D_EMB, 8 * FC_HIDDEN), lambda i: (0, 0)),
            pl.BlockSpec((8, 8 * FC_HIDDEN), lambda i: (0, 0)),
            pl.BlockSpec((8 * FC_HIDDEN, 8 * D), lambda i: (0, 0)),
            pl.BlockSpec((D, D), lambda i: (0, 0)),
        ],
        out_specs=pl.BlockSpec((EBLK, D), lambda i: (i, 0)),
        out_shape=jax.ShapeDtypeStruct((nblk * EBLK, D), jnp.float32),
    )(g, embp, eap8, bd1, exp8, bd2, w2)


_PBLK = np.arange(EBLK).reshape(RPB, 8).T.reshape(-1)
_EPERM = (np.arange(0, E, EBLK)[:, None] + _PBLK[None, :]).reshape(-1)


def _scatter_call(n_edges):
    per_w = n_edges // NUM_WORKERS
    nchunk = per_w // CHUNK
    ngroup = nchunk // K2
    tail = nchunk - ngroup * K2

    @functools.partial(
        pl.kernel,
        out_type=jax.ShapeDtypeStruct((NUM_SC, NP, D), jnp.float32),
        mesh=_MESH,
        scratch_types=[
            pltpu.VMEM((CHUNK,), jnp.int32),
            pltpu.VMEM((CHUNK,), jnp.int32),
            pltpu.VMEM((CHUNK,), jnp.int32),
            pltpu.VMEM((CHUNK,), jnp.int32),
            pltpu.VMEM((K2, CHUNK, D), jnp.float32),
            pltpu.VMEM_SHARED((NP, D), jnp.float32),
            pltpu.SemaphoreType.DMA,
        ],
    )
    def kfn(msg_hbm, dst_hbm, zeros_hbm, out_hbm,
            i0, i1, i2, i3, rows_v, acc_sh, lsem):
        idx_bufs = (i0, i1, i2, i3)
        cid = lax.axis_index("c")
        sid = lax.axis_index("s")
        wid = cid * NUM_TILES + sid
        base = wid * per_w

        pltpu.sync_copy(zeros_hbm, acc_sh.at[pl.ds(sid * STRIPE, STRIPE)])
        plsc.subcore_barrier()

        @pl.loop(0, ngroup)
        def _(g):
            goff = base + g * (K2 * CHUNK)
            hs = []
            for b in range(K2):
                hs.append(pltpu.async_copy(
                    dst_hbm.at[pl.ds(goff + b * CHUNK, CHUNK)],
                    idx_bufs[b], lsem))
                hs.append(pltpu.async_copy(
                    msg_hbm.at[pl.ds(goff + b * CHUNK, CHUNK)],
                    rows_v.at[b], lsem))
            for h in hs:
                h.wait()
            for b in range(K2):
                pltpu.sync_copy(rows_v.at[b], acc_sh.at[idx_bufs[b]],
                                add=True)

        for t in range(tail):
            toff = base + (ngroup * K2 + t) * CHUNK
            pltpu.sync_copy(dst_hbm.at[pl.ds(toff, CHUNK)], i0)
            pltpu.sync_copy(msg_hbm.at[pl.ds(toff, CHUNK)], rows_v.at[0])
            pltpu.sync_copy(rows_v.at[0], acc_sh.at[i0], add=True)

        plsc.subcore_barrier()
        pltpu.sync_copy(acc_sh.at[pl.ds(sid * STRIPE, STRIPE)],
                        out_hbm.at[cid, pl.ds(sid * STRIPE, STRIPE)])

    return kfn


_sc_scatter_a = _scatter_call(EA_EDGES)
_sc_scatter_b = _scatter_call(EB_EDGES)


def _final_body(pa_ref, pb_ref, natp_ref, sc_ref, w3_ref, out_ref):
    agg = (pa_ref[0] + pa_ref[1] + pb_ref[0] + pb_ref[1]) * _INV_SQRT_NEIGH
    upd = jax.nn.silu(
        jnp.dot(agg * natp_ref[...], w3_ref[...],
                preferred_element_type=jnp.float32) * _INV_SQRT_D)
    out_ref[...] = upd + sc_ref[...]


def _final(pa, pb, natp, sc, w3):
    blk = 400
    grid = (N // blk,)
    return pl.pallas_call(
        _final_body,
        grid=grid,
        in_specs=[
            pl.BlockSpec((NUM_SC, blk, D), lambda i: (0, i, 0)),
            pl.BlockSpec((NUM_SC, blk, D), lambda i: (0, i, 0)),
            pl.BlockSpec((blk, D), lambda i: (i, 0)),
            pl.BlockSpec((blk, D), lambda i: (i, 0)),
            pl.BlockSpec((D, D), lambda i: (0, 0)),
        ],
        out_specs=pl.BlockSpec((blk, D), lambda i: (i, 0)),
        out_shape=jax.ShapeDtypeStruct((N, D), jnp.float32),
    )(pa, pb, natp, sc, w3)


def kernel(node_feats, node_attrs, edge_embedding, edge_attrs, edge_index,
           W1, M1, M2, W2, Wtp, W3, Wsc):
    perm = jnp.asarray(_EPERM)
    src = jnp.take(edge_index[0], perm)
    dst = jnp.take(edge_index[1], perm)
    wtpt = Wtp.T
    wsct = jnp.transpose(Wsc, (2, 1, 0))

    eap8 = edge_attrs.reshape(E // 8, 8)
    eye8 = jnp.eye(8, dtype=jnp.float32)
    bd1 = (eye8[:, None, :, None] * M1[None, :, None, :]).reshape(
        8 * D_EMB, 8 * FC_HIDDEN)
    exp8 = (eye8[:, :, None] * jnp.ones((1, 1, FC_HIDDEN))).reshape(
        8, 8 * FC_HIDDEN)
    bd2 = (eye8[:, None, :, None] * M2[None, :, None, :]).reshape(
        8 * FC_HIDDEN, 8 * D)

    x, natp, sc = _node_prep(node_feats, node_attrs, W1, wtpt, wsct)

    embp = edge_embedding.reshape(E // 8, 8 * D_EMB)
    g = _sc_gather(x, src)

    nblk_a = EA_EDGES // EBLK
    nblk_b = EB_EDGES // EBLK
    msg_a = _edge_stage(g, embp, eap8, bd1, exp8, bd2, W2, 0, nblk_a)
    msg_b = _edge_stage(g, embp, eap8, bd1, exp8, bd2, W2, nblk_a, nblk_b)

    dst_a = lax.slice_in_dim(dst, 0, EA_EDGES)
    dst_b = lax.slice_in_dim(dst, EA_EDGES, E)
    zeros = jnp.zeros((STRIPE, D), jnp.float32)
    pa = _sc_scatter_a(msg_a, dst_a, zeros)
    pb = _sc_scatter_b(msg_b, dst_b, zeros)
    return _final(pa, pb, natp, sc, W3)


# speedup vs baseline: 1.1586x; 1.0084x over previous
import functools
import math

import numpy as np

import jax
import jax.numpy as jnp
from jax import lax
from jax.experimental import pallas as pl
from jax.experimental.pallas import tpu as pltpu
from jax.experimental.pallas import tpu_sc as plsc

N = 10000
E = 320000
D = 128
D_ATTR = 16
D_EMB = 16
FC_HIDDEN = 8

NUM_SC = 2
NUM_TILES = 16
NUM_WORKERS = NUM_SC * NUM_TILES

PER_W = E // NUM_WORKERS
CHUNK = 80
NCHUNK = PER_W // CHUNK
K = 5
NGROUP = NCHUNK // K
K2 = 4

EA_EDGES = 192000
EB_EDGES = 128000

NP = 10240
STRIPE = NP // NUM_TILES

_INV_SQRT_D = 1.0 / math.sqrt(D)
_INV_SQRT_DA = 1.0 / math.sqrt(D_ATTR)
_INV_SQRT_DE = 1.0 / math.sqrt(D_EMB)
_INV_SQRT_FC = 1.0 / math.sqrt(FC_HIDDEN)
_INV_SQRT_NEIGH = 1.0 / math.sqrt(32.0)
_INV_SQRT_DDA = 1.0 / math.sqrt(D * D_ATTR)

_MESH = plsc.VectorSubcoreMesh(
    core_axis_name="c", subcore_axis_name="s",
    num_cores=NUM_SC, num_subcores=NUM_TILES)


def _node_prep_body(nf_ref, na_ref, w1_ref, wtpt_ref, wsct_ref,
                    x_ref, natp_ref, sc_ref):
    nf = nf_ref[...]
    na = na_ref[...]
    x_ref[...] = jnp.dot(nf, w1_ref[...],
                         preferred_element_type=jnp.float32) * _INV_SQRT_D
    natp_ref[...] = jnp.dot(na, wtpt_ref[...],
                            preferred_element_type=jnp.float32) * _INV_SQRT_DA
    acc = jnp.zeros_like(nf)
    for j in range(D_ATTR):
        acc = acc + jnp.dot(nf * na[:, j:j + 1], wsct_ref[j],
                            preferred_element_type=jnp.float32)
    sc_ref[...] = acc * _INV_SQRT_DDA


def _node_prep(nf, na, w1, wtpt, wsct):
    blk = 400
    grid = (N // blk,)
    return pl.pallas_call(
        _node_prep_body,
        grid=grid,
        in_specs=[
            pl.BlockSpec((blk, D), lambda i: (i, 0)),
            pl.BlockSpec((blk, D_ATTR), lambda i: (i, 0)),
            pl.BlockSpec((D, D), lambda i: (0, 0)),
            pl.BlockSpec((D_ATTR, D), lambda i: (0, 0)),
            pl.BlockSpec((D_ATTR, D, D), lambda i: (0, 0, 0)),
        ],
        out_specs=[
            pl.BlockSpec((blk, D), lambda i: (i, 0)),
            pl.BlockSpec((blk, D), lambda i: (i, 0)),
            pl.BlockSpec((blk, D), lambda i: (i, 0)),
        ],
        out_shape=[
            jax.ShapeDtypeStruct((N, D), jnp.float32),
            jax.ShapeDtypeStruct((N, D), jnp.float32),
            jax.ShapeDtypeStruct((N, D), jnp.float32),
        ],
    )(nf, na, w1, wtpt, wsct)


def _gather_call(n_edges):
    per_w = n_edges // NUM_WORKERS
    ngroup = per_w // CHUNK // K

    @functools.partial(
        pl.kernel,
        out_type=jax.ShapeDtypeStruct((n_edges, D), jnp.float32),
        mesh=_MESH,
        scratch_types=[
            pltpu.VMEM((per_w,), jnp.int32),
            pltpu.VMEM((K, CHUNK, D), jnp.float32),
            pltpu.SemaphoreType.DMA,
            pltpu.SemaphoreType.DMA,
        ],
    )
    def kfn(x_hbm, src_hbm, out_hbm, idx_v, bufs_v, gsem, wsem):
        wid = lax.axis_index("c") * NUM_TILES + lax.axis_index("s")
        base = wid * per_w
        pltpu.sync_copy(src_hbm.at[pl.ds(base, per_w)], idx_v)

        @pl.loop(0, ngroup)
        def _(g):
            goff = g * (K * CHUNK)
            hs = [
                pltpu.async_copy(
                    x_hbm.at[idx_v.at[pl.ds(goff + b * CHUNK, CHUNK)]],
                    bufs_v.at[b], gsem)
                for b in range(K)
            ]
            for h in hs:
                h.wait()
            ws = [
                pltpu.async_copy(
                    bufs_v.at[b],
                    out_hbm.at[pl.ds(base + goff + b * CHUNK, CHUNK)], wsem)
                for b in range(K)
            ]
            for h in ws:
                h.wait()

    return kfn


_sc_gather_a = _gather_call(EA_EDGES)
_sc_gather_b = _gather_call(EB_EDGES)


EBLK = 3200
RPB = EBLK // 8


def _edge_body(g_ref, emb_ref, eap_ref, bd1_ref, exp8_ref, bd2_ref, w2_ref,
               msg_ref):
    hp = jax.nn.silu(jnp.dot(emb_ref[...], bd1_ref[...],
                             preferred_element_type=jnp.float32)
                     * _INV_SQRT_DE)
    eap64 = jnp.dot(eap_ref[...], exp8_ref[...],
                    preferred_element_type=jnp.float32)
    cq = jnp.dot(hp * eap64, bd2_ref[...],
                 preferred_element_type=jnp.float32) * _INV_SQRT_FC
    for k in range(8):
        ck = cq[:, 128 * k:128 * (k + 1)]
        mk = g_ref[RPB * k:RPB * (k + 1), :] * ck
        msg_ref[RPB * k:RPB * (k + 1), :] = jax.nn.silu(
            jnp.dot(mk, w2_ref[...],
                    preferred_element_type=jnp.float32) * _INV_SQRT_D)


def _edge_stage(g, embp, eap8, bd1, exp8, bd2, w2, blk0, nblk):
    return pl.pallas_call(
        _edge_body,
        grid=(nblk,),
        in_specs=[
            pl.BlockSpec((EBLK, D), lambda i: (i, 0)),
            pl.BlockSpec((RPB, 8 * D_EMB), lambda i: (i, 0)),
            pl.BlockSpec((RPB, 8), lambda i: (i + blk0, 0)),
            pl.BlockSpec((8 * D_EMB, 8 * FC_HIDDEN), lambda i: (0, 0)),
            pl.BlockSpec((8, 8 * FC_HIDDEN), lambda i: (0, 0)),
            pl.BlockSpec((8 * FC_HIDDEN, 8 * D), lambda i: (0, 0)),
            pl.BlockSpec((D, D), lambda i: (0, 0)),
        ],
        out_specs=pl.BlockSpec((EBLK, D), lambda i: (i, 0)),
        out_shape=jax.ShapeDtypeStruct((nblk * EBLK, D), jnp.float32),
    )(g, embp, eap8, bd1, exp8, bd2, w2)


_PBLK = np.arange(EBLK).reshape(RPB, 8).T.reshape(-1)
_EPERM = (np.arange(0, E, EBLK)[:, None] + _PBLK[None, :]).reshape(-1)


def _scatter_call(n_edges):
    per_w = n_edges // NUM_WORKERS
    nchunk = per_w // CHUNK
    ngroup = nchunk // K2
    tail = nchunk - ngroup * K2

    @functools.partial(
        pl.kernel,
        out_type=jax.ShapeDtypeStruct((NUM_SC, NP, D), jnp.float32),
        mesh=_MESH,
        scratch_types=[
            pltpu.VMEM((CHUNK,), jnp.int32),
            pltpu.VMEM((CHUNK,), jnp.int32),
            pltpu.VMEM((CHUNK,), jnp.int32),
            pltpu.VMEM((CHUNK,), jnp.int32),
            pltpu.VMEM((K2, CHUNK, D), jnp.float32),
            pltpu.VMEM_SHARED((NP, D), jnp.float32),
            pltpu.SemaphoreType.DMA,
        ],
    )
    def kfn(msg_hbm, dst_hbm, zeros_hbm, out_hbm,
            i0, i1, i2, i3, rows_v, acc_sh, lsem):
        idx_bufs = (i0, i1, i2, i3)
        cid = lax.axis_index("c")
        sid = lax.axis_index("s")
        wid = cid * NUM_TILES + sid
        base = wid * per_w

        pltpu.sync_copy(zeros_hbm, acc_sh.at[pl.ds(sid * STRIPE, STRIPE)])
        plsc.subcore_barrier()

        @pl.loop(0, ngroup)
        def _(g):
            goff = base + g * (K2 * CHUNK)
            hs = []
            for b in range(K2):
                hs.append(pltpu.async_copy(
                    dst_hbm.at[pl.ds(goff + b * CHUNK, CHUNK)],
                    idx_bufs[b], lsem))
                hs.append(pltpu.async_copy(
                    msg_hbm.at[pl.ds(goff + b * CHUNK, CHUNK)],
                    rows_v.at[b], lsem))
            for h in hs:
                h.wait()
            for b in range(K2):
                pltpu.sync_copy(rows_v.at[b], acc_sh.at[idx_bufs[b]],
                                add=True)

        for t in range(tail):
            toff = base + (ngroup * K2 + t) * CHUNK
            pltpu.sync_copy(dst_hbm.at[pl.ds(toff, CHUNK)], i0)
            pltpu.sync_copy(msg_hbm.at[pl.ds(toff, CHUNK)], rows_v.at[0])
            pltpu.sync_copy(rows_v.at[0], acc_sh.at[i0], add=True)

        plsc.subcore_barrier()
        pltpu.sync_copy(acc_sh.at[pl.ds(sid * STRIPE, STRIPE)],
                        out_hbm.at[cid, pl.ds(sid * STRIPE, STRIPE)])

    return kfn


_sc_scatter_a = _scatter_call(EA_EDGES)
_sc_scatter_b = _scatter_call(EB_EDGES)


def _final_body(pa_ref, pb_ref, natp_ref, sc_ref, w3_ref, out_ref):
    agg = (pa_ref[0] + pa_ref[1] + pb_ref[0] + pb_ref[1]) * _INV_SQRT_NEIGH
    upd = jax.nn.silu(
        jnp.dot(agg * natp_ref[...], w3_ref[...],
                preferred_element_type=jnp.float32) * _INV_SQRT_D)
    out_ref[...] = upd + sc_ref[...]


def _final(pa, pb, natp, sc, w3):
    blk = 400
    grid = (N // blk,)
    return pl.pallas_call(
        _final_body,
        grid=grid,
        in_specs=[
            pl.BlockSpec((NUM_SC, blk, D), lambda i: (0, i, 0)),
            pl.BlockSpec((NUM_SC, blk, D), lambda i: (0, i, 0)),
            pl.BlockSpec((blk, D), lambda i: (i, 0)),
            pl.BlockSpec((blk, D), lambda i: (i, 0)),
            pl.BlockSpec((D, D), lambda i: (0, 0)),
        ],
        out_specs=pl.BlockSpec((blk, D), lambda i: (i, 0)),
        out_shape=jax.ShapeDtypeStruct((N, D), jnp.float32),
    )(pa, pb, natp, sc, w3)


def kernel(node_feats, node_attrs, edge_embedding, edge_attrs, edge_index,
           W1, M1, M2, W2, Wtp, W3, Wsc):
    perm = jnp.asarray(_EPERM)
    src = jnp.take(edge_index[0], perm)
    dst = jnp.take(edge_index[1], perm)
    wtpt = Wtp.T
    wsct = jnp.transpose(Wsc, (2, 1, 0))

    eap8 = edge_attrs.reshape(E // 8, 8)
    eye8 = jnp.eye(8, dtype=jnp.float32)
    bd1 = (eye8[:, None, :, None] * M1[None, :, None, :]).reshape(
        8 * D_EMB, 8 * FC_HIDDEN)
    exp8 = (eye8[:, :, None] * jnp.ones((1, 1, FC_HIDDEN))).reshape(
        8, 8 * FC_HIDDEN)
    bd2 = (eye8[:, None, :, None] * M2[None, :, None, :]).reshape(
        8 * FC_HIDDEN, 8 * D)

    x, natp, sc = _node_prep(node_feats, node_attrs, W1, wtpt, wsct)

    nblk_a = EA_EDGES // EBLK
    nblk_b = EB_EDGES // EBLK
    src_a = lax.slice_in_dim(src, 0, EA_EDGES)
    src_b = lax.slice_in_dim(src, EA_EDGES, E)
    embp_a = lax.slice_in_dim(edge_embedding, 0, EA_EDGES).reshape(
        EA_EDGES // 8, 8 * D_EMB)
    embp_b = lax.slice_in_dim(edge_embedding, EA_EDGES, E).reshape(
        EB_EDGES // 8, 8 * D_EMB)

    g_a = _sc_gather_a(x, src_a)
    g_b = _sc_gather_b(x, src_b)
    msg_a = _edge_stage(g_a, embp_a, eap8, bd1, exp8, bd2, W2, 0, nblk_a)
    msg_b = _edge_stage(g_b, embp_b, eap8, bd1, exp8, bd2, W2, nblk_a,
                        nblk_b)

    dst_a = lax.slice_in_dim(dst, 0, EA_EDGES)
    dst_b = lax.slice_in_dim(dst, EA_EDGES, E)
    zeros = jnp.zeros((STRIPE, D), jnp.float32)
    pa = _sc_scatter_a(msg_a, dst_a, zeros)
    pb = _sc_scatter_b(msg_b, dst_b, zeros)
    return _final(pa, pb, natp, sc, W3)
